# baseline (speedup 1.0000x reference)
import functools

import jax
import jax.numpy as jnp
from jax import lax
from jax.experimental import pallas as pl
from jax.experimental.pallas import tpu as pltpu
from jax.experimental.pallas import tpu_sc as plsc

D_MODEL = 1024
N_BATCH = 4
SEQ = 4096
N_TOK = N_BATCH * SEQ
N_WORKERS = 32
S_PER_W = SEQ // N_WORKERS
SS = 8
SROWS = N_BATCH * SS
N_SUPER = S_PER_W // SS
NB = 3
LANES = 16
PER_W = N_BATCH * S_PER_W


@functools.partial(
    pl.kernel,
    mesh=plsc.VectorSubcoreMesh(core_axis_name="c", subcore_axis_name="s"),
    out_type=jax.ShapeDtypeStruct((N_TOK, D_MODEL), jnp.float32),
    scratch_types=(
        [pltpu.VMEM((PER_W,), jnp.int32),
         pltpu.VMEM((PER_W,), jnp.int32),
         pltpu.VMEM((N_SUPER, SROWS), jnp.int32),
         pltpu.VMEM((NB * SROWS, D_MODEL), jnp.float32),
         pltpu.VMEM((SS, D_MODEL), jnp.float32),
         pltpu.VMEM((SS, D_MODEL), jnp.float32),
         pltpu.VMEM((SS, D_MODEL), jnp.float32)]
        + [pltpu.SemaphoreType.DMA for _ in range(3 * NB)]
    ),
    compiler_params=pltpu.CompilerParams(needs_layout_passes=False),
)
def _embed_sc(x_hbm, table_hbm, pos_hbm, out_hbm,
              stage_v, idx_v, out_idx, rows_v, pos0, pos1, pos2,
              g0, g1, g2, p0, p1, p2, o0, o1, o2):
    pos_v = (pos0, pos1, pos2)
    g_sem = (g0, g1, g2)
    p_sem = (p0, p1, p2)
    o_sem = (o0, o1, o2)

    wid = lax.axis_index("s") * 2 + lax.axis_index("c")
    s_base = wid * S_PER_W

    io = lax.iota(jnp.int32, LANES)
    lane_b = lax.shift_right_logical(io, 2)
    lane_s = jnp.bitwise_and(io, 3)
    lane_off = lane_b * S_PER_W + lane_s
    out_lane = lane_b * SEQ + s_base + lane_s

    def issue_pos(g, slot):
        pltpu.async_copy(pos_hbm.at[pl.ds(s_base + g * SS, SS)],
                         pos_v[slot], p_sem[slot])

    def issue_gather(g, slot):
        pltpu.async_copy(
            table_hbm.at[idx_v.at[pl.ds(g * SROWS, SROWS)]],
            rows_v.at[pl.ds(slot * SROWS, SROWS)], g_sem[slot])

    def wait_pos(slot):
        pltpu.make_async_copy(pos_hbm.at[pl.ds(0, SS)],
                              pos_v[slot], p_sem[slot]).wait()

    def wait_gather(slot):
        pltpu.make_async_copy(
            table_hbm.at[idx_v.at[pl.ds(0, SROWS)]],
            rows_v.at[pl.ds(slot * SROWS, SROWS)], g_sem[slot]).wait()

    def wait_scatter(slot):
        pltpu.make_async_copy(
            rows_v.at[pl.ds(slot * SROWS, SROWS)],
            out_hbm.at[out_idx.at[0]], o_sem[slot]).wait()

    issue_pos(0, 0)
    issue_pos(1, 1)
    stage_cps = [
        pltpu.async_copy(x_hbm.at[pl.ds(bi * SEQ + s_base, S_PER_W)],
                         stage_v.at[pl.ds(bi * S_PER_W, S_PER_W)],
                         o_sem[0])
        for bi in range(N_BATCH)]
    for cp in stage_cps:
        cp.wait()
    for t in range(PER_W // LANES):
        idx_v[pl.ds(t * LANES, LANES)] = plsc.load_gather(
            stage_v, [lane_off + t * N_BATCH])
        out_idx[t // 2, pl.ds((t % 2) * LANES, LANES)] = (
            out_lane + t * N_BATCH)
    issue_gather(0, 0)
    issue_gather(1, 1)

    def super_body(g, slot, prefetch):
        wait_gather(slot)
        wait_pos(slot)

        def add_body(j, jcarry):
            for u in range(2):
                sl = pl.ds((j * 2 + u) * LANES, LANES)
                for s in range(SS):
                    pv = pos_v[slot][s, sl]
                    for bi in range(N_BATCH):
                        r = slot * SROWS + (s // 4) * LANES + bi * 4 + (s % 4)
                        rows_v[r, sl] = rows_v[r, sl] + pv
            return jcarry
        lax.fori_loop(0, D_MODEL // LANES // 2, add_body, 0)

        pltpu.async_copy(rows_v.at[pl.ds(slot * SROWS, SROWS)],
                         out_hbm.at[out_idx.at[g]], o_sem[slot])

        if prefetch:
            nslot = (slot + 2) % NB
            @pl.when(g + 2 < N_SUPER)
            def _():
                @pl.when(g >= 1)
                def _():
                    wait_scatter(nslot)
                issue_gather(g + 2, nslot)
                issue_pos(g + 2, nslot)

    def outer(it, carry):
        for k in range(NB):
            super_body(it * NB + k, k, True)
        return carry

    lax.fori_loop(0, (N_SUPER - 1) // NB, outer, 0)
    super_body(N_SUPER - 1, (N_SUPER - 1) % NB, False)

    for g in range(N_SUPER - 3, N_SUPER):
        wait_scatter(g % NB)


def kernel(x, table, pos_encoding):
    out = _embed_sc(x.reshape(-1).astype(jnp.int32), table, pos_encoding)
    return out.reshape(N_BATCH, SEQ, D_MODEL)


# speedup vs baseline: 2.3658x; 2.3658x over previous
import functools

import jax
import jax.numpy as jnp
from jax import lax
from jax.experimental import pallas as pl
from jax.experimental.pallas import tpu as pltpu
from jax.experimental.pallas import tpu_sc as plsc

D_MODEL = 1024
N_BATCH = 4
SEQ = 4096
N_TOK = N_BATCH * SEQ
N_WORKERS = 32
S_PER_W = SEQ // N_WORKERS
SS = 8
SROWS = N_BATCH * SS
N_SUPER = S_PER_W // SS
NB = 3
LANES = 16
PER_W = N_BATCH * S_PER_W


@functools.partial(
    pl.kernel,
    mesh=plsc.VectorSubcoreMesh(core_axis_name="c", subcore_axis_name="s"),
    out_type=jax.ShapeDtypeStruct((N_TOK, D_MODEL), jnp.float32),
    scratch_types=(
        [pltpu.VMEM((PER_W,), jnp.int32),
         pltpu.VMEM((PER_W,), jnp.int32),
         pltpu.VMEM((N_SUPER, SROWS), jnp.int32),
         pltpu.VMEM((NB * SROWS, D_MODEL), jnp.float32),
         pltpu.VMEM((SS, D_MODEL), jnp.float32),
         pltpu.VMEM((SS, D_MODEL), jnp.float32),
         pltpu.VMEM((SS, D_MODEL), jnp.float32)]
        + [pltpu.SemaphoreType.DMA for _ in range(3 * NB)]
    ),
    compiler_params=pltpu.CompilerParams(needs_layout_passes=False),
)
def _embed_sc(x_hbm, table_hbm, pos_hbm, out_hbm,
              stage_v, idx_v, out_idx, rows_v, pos0, pos1, pos2,
              g0, g1, g2, p0, p1, p2, o0, o1, o2):
    pos_v = (pos0, pos1, pos2)
    g_sem = (g0, g1, g2)
    p_sem = (p0, p1, p2)
    o_sem = (o0, o1, o2)

    wid = lax.axis_index("s") * 2 + lax.axis_index("c")
    s_base = wid * S_PER_W

    io = lax.iota(jnp.int32, LANES)
    lane_b = lax.shift_right_logical(io, 2)
    lane_s = jnp.bitwise_and(io, 3)
    lane_off = lane_b * S_PER_W + lane_s
    out_lane = lane_b * SEQ + s_base + lane_s

    def issue_pos(g, slot):
        pltpu.async_copy(pos_hbm.at[pl.ds(s_base + g * SS, SS)],
                         pos_v[slot], p_sem[slot])

    def issue_gather(g, slot):
        pltpu.async_copy(
            table_hbm.at[idx_v.at[pl.ds(g * SROWS, SROWS)]],
            rows_v.at[pl.ds(slot * SROWS, SROWS)], g_sem[slot])

    def wait_pos(slot):
        pltpu.make_async_copy(pos_hbm.at[pl.ds(0, SS)],
                              pos_v[slot], p_sem[slot]).wait()

    def wait_gather(slot):
        pltpu.make_async_copy(
            table_hbm.at[idx_v.at[pl.ds(0, SROWS)]],
            rows_v.at[pl.ds(slot * SROWS, SROWS)], g_sem[slot]).wait()

    def wait_scatter(slot):
        pltpu.make_async_copy(
            rows_v.at[pl.ds(slot * SROWS, SROWS)],
            out_hbm.at[out_idx.at[0]], o_sem[slot]).wait()

    issue_pos(0, 0)
    issue_pos(1, 1)
    stage_cps = [
        pltpu.async_copy(x_hbm.at[pl.ds(bi * SEQ + s_base, S_PER_W)],
                         stage_v.at[pl.ds(bi * S_PER_W, S_PER_W)],
                         o_sem[0])
        for bi in range(N_BATCH)]
    for cp in stage_cps:
        cp.wait()
    for t in range(PER_W // LANES):
        idx_v[pl.ds(t * LANES, LANES)] = plsc.load_gather(
            stage_v, [lane_off + t * N_BATCH])
        out_idx[t // 2, pl.ds((t % 2) * LANES, LANES)] = (
            out_lane + t * N_BATCH)
    issue_gather(0, 0)
    issue_gather(1, 1)

    def super_body(g, slot, prefetch):
        wait_gather(slot)
        wait_pos(slot)

        def add_body(j, jcarry):
            sl = pl.ds(pl.multiple_of(j * LANES, LANES), LANES)
            for s in range(SS):
                pv = pos_v[slot][s, sl]
                for bi in range(N_BATCH):
                    r = slot * SROWS + (s // 4) * LANES + bi * 4 + (s % 4)
                    plsc.addupdate(rows_v.at[r, sl], pv)
            return jcarry
        lax.fori_loop(0, D_MODEL // LANES, add_body, 0)

        pltpu.async_copy(rows_v.at[pl.ds(slot * SROWS, SROWS)],
                         out_hbm.at[out_idx.at[g]], o_sem[slot])

        if prefetch:
            nslot = (slot + 2) % NB
            @pl.when(g + 2 < N_SUPER)
            def _():
                @pl.when(g >= 1)
                def _():
                    wait_scatter(nslot)
                issue_gather(g + 2, nslot)
                issue_pos(g + 2, nslot)

    def outer(it, carry):
        for k in range(NB):
            super_body(it * NB + k, k, True)
        return carry

    lax.fori_loop(0, (N_SUPER - 1) // NB, outer, 0)
    super_body(N_SUPER - 1, (N_SUPER - 1) % NB, False)

    for g in range(N_SUPER - 3, N_SUPER):
        wait_scatter(g % NB)


def kernel(x, table, pos_encoding):
    out = _embed_sc(x.reshape(-1).astype(jnp.int32), table, pos_encoding)
    return out.reshape(N_BATCH, SEQ, D_MODEL)


# speedup vs baseline: 2.3881x; 1.0094x over previous
import functools

import jax
import jax.numpy as jnp
from jax import lax
from jax.experimental import pallas as pl
from jax.experimental.pallas import tpu as pltpu
from jax.experimental.pallas import tpu_sc as plsc

D_MODEL = 1024
N_BATCH = 4
SEQ = 4096
N_TOK = N_BATCH * SEQ
N_WORKERS = 32
S_PER_W = SEQ // N_WORKERS
SS = 8
SROWS = N_BATCH * SS
N_SUPER = S_PER_W // SS
NB = 3
LANES = 16
PER_W = N_BATCH * S_PER_W


@functools.partial(
    pl.kernel,
    mesh=plsc.VectorSubcoreMesh(core_axis_name="c", subcore_axis_name="s"),
    out_type=jax.ShapeDtypeStruct((N_TOK, D_MODEL), jnp.float32),
    scratch_types=(
        [pltpu.VMEM((PER_W,), jnp.int32),
         pltpu.VMEM((PER_W,), jnp.int32),
         pltpu.VMEM((N_SUPER, SROWS), jnp.int32),
         pltpu.VMEM((NB * SROWS, D_MODEL), jnp.float32),
         pltpu.VMEM((SS, D_MODEL), jnp.float32),
         pltpu.VMEM((SS, D_MODEL), jnp.float32),
         pltpu.VMEM((SS, D_MODEL), jnp.float32)]
        + [pltpu.SemaphoreType.DMA for _ in range(3 * NB)]
    ),
    compiler_params=pltpu.CompilerParams(needs_layout_passes=False),
)
def _embed_sc(x_hbm, table_hbm, pos_hbm, out_hbm,
              stage_v, idx_v, out_idx, rows_v, pos0, pos1, pos2,
              g0, g1, g2, p0, p1, p2, o0, o1, o2):
    pos_v = (pos0, pos1, pos2)
    g_sem = (g0, g1, g2)
    p_sem = (p0, p1, p2)
    o_sem = (o0, o1, o2)

    wid = lax.axis_index("s") * 2 + lax.axis_index("c")
    s_base = wid * S_PER_W

    io = lax.iota(jnp.int32, LANES)
    lane_b = lax.shift_right_logical(io, 2)
    lane_s = jnp.bitwise_and(io, 3)
    lane_off = lane_b * S_PER_W + lane_s
    out_lane = lane_b * SEQ + s_base + lane_s

    def issue_pos(g, slot):
        pltpu.async_copy(pos_hbm.at[pl.ds(s_base + g * SS, SS)],
                         pos_v[slot], p_sem[slot])

    def issue_gather(g, slot):
        pltpu.async_copy(
            table_hbm.at[idx_v.at[pl.ds(g * SROWS, SROWS)]],
            rows_v.at[pl.ds(slot * SROWS, SROWS)], g_sem[slot])

    def wait_pos(slot):
        pltpu.make_async_copy(pos_hbm.at[pl.ds(0, SS)],
                              pos_v[slot], p_sem[slot]).wait()

    def wait_gather(slot):
        pltpu.make_async_copy(
            table_hbm.at[idx_v.at[pl.ds(0, SROWS)]],
            rows_v.at[pl.ds(slot * SROWS, SROWS)], g_sem[slot]).wait()

    def wait_scatter(slot):
        pltpu.make_async_copy(
            rows_v.at[pl.ds(slot * SROWS, SROWS)],
            out_hbm.at[out_idx.at[0]], o_sem[slot]).wait()

    issue_pos(0, 0)
    issue_pos(1, 1)
    stage_cps = [
        pltpu.async_copy(x_hbm.at[pl.ds(bi * SEQ + s_base, S_PER_W)],
                         stage_v.at[pl.ds(bi * S_PER_W, S_PER_W)],
                         o_sem[0])
        for bi in range(N_BATCH)]
    for cp in stage_cps:
        cp.wait()
    for t in range(PER_W // LANES):
        idx_v[pl.ds(t * LANES, LANES)] = plsc.load_gather(
            stage_v, [lane_off + t * N_BATCH])
        out_idx[t // 2, pl.ds((t % 2) * LANES, LANES)] = (
            out_lane + t * N_BATCH)
    issue_gather(0, 0)
    issue_gather(1, 1)

    def super_body(g, slot, prefetch):
        wait_gather(slot)
        wait_pos(slot)
        if prefetch:
            @pl.when(g + 2 < N_SUPER)
            def _():
                issue_pos(g + 2, (slot + 2) % NB)

        def add_body(j, jcarry):
            sl = pl.ds(pl.multiple_of(j * LANES, LANES), LANES)
            for s in range(SS):
                pv = pos_v[slot][s, sl]
                for bi in range(N_BATCH):
                    r = slot * SROWS + (s // 4) * LANES + bi * 4 + (s % 4)
                    plsc.addupdate(rows_v.at[r, sl], pv)
            return jcarry
        lax.fori_loop(0, D_MODEL // LANES, add_body, 0)

        pltpu.async_copy(rows_v.at[pl.ds(slot * SROWS, SROWS)],
                         out_hbm.at[out_idx.at[g]], o_sem[slot])

        if prefetch:
            nslot = (slot + 2) % NB
            @pl.when(g + 2 < N_SUPER)
            def _():
                @pl.when(g >= 1)
                def _():
                    wait_scatter(nslot)
                issue_gather(g + 2, nslot)

    def outer(it, carry):
        for k in range(NB):
            super_body(it * NB + k, k, True)
        return carry

    lax.fori_loop(0, (N_SUPER - 1) // NB, outer, 0)
    super_body(N_SUPER - 1, (N_SUPER - 1) % NB, False)

    for g in range(N_SUPER - 3, N_SUPER):
        wait_scatter(g % NB)


def kernel(x, table, pos_encoding):
    out = _embed_sc(x.reshape(-1).astype(jnp.int32), table, pos_encoding)
    return out.reshape(N_BATCH, SEQ, D_MODEL)
